# baseline (speedup 1.0000x reference)
import functools

import jax
import jax.numpy as jnp
import numpy as np
from jax.experimental import pallas as pl
from jax.experimental.pallas import tpu as pltpu

B = 128
OBS_LEN = 256
MEM_LEN = 256
NUM_MEMS = 4096
EMBED = 64
HEADS = 4
HEAD_DIM = EMBED // HEADS
K = 10
OUT_DIM = 64

_NEG = np.float32(-3.0e38)
_POS = np.float32(3.0e38)

NCHUNK = 2
CHUNK = NUM_MEMS // NCHUNK


def _body(obs_ref, memhead_ref, memtail_ref, W_obs_ref, b_obs_ref,
          Wq_ref, bq_ref, Wk_ref, bk_ref, Wv_ref, bv_ref,
          Wo_ref, bo_ref, W_out_ref, b_out_ref, out_ref,
          s1_ref, s2_ref, vall_ref, s_ref, wc_ref, bc_ref):
    f32 = jnp.float32
    step = pl.program_id(0)
    dot = functools.partial(jnp.dot, preferred_element_type=f32)

    mh = memhead_ref[...]
    p1 = jnp.sum(mh.reshape(CHUNK // 8, 8, OBS_LEN), axis=0)
    p2 = jnp.sum((mh * mh).reshape(CHUNK // 8, 8, OBS_LEN), axis=0)

    @pl.when(step == 0)
    def _first():
        s1_ref[...] = p1
        s2_ref[...] = p2
        mt = memtail_ref[...]
        Kall = dot(mt, Wk_ref[...].T) + bk_ref[...]
        vall_ref[...] = dot(mt, Wv_ref[...].T) + bv_ref[...]
        obs_emb = dot(obs_ref[...], W_obs_ref[...].T) + b_obs_ref[...]
        q = dot(obs_emb, Wq_ref[...].T) + bq_ref[...]
        hrow = jax.lax.broadcasted_iota(jnp.int32, (HEADS * B, EMBED), 0)
        hcol = jax.lax.broadcasted_iota(jnp.int32, (HEADS * B, EMBED), 1)
        hm = ((hrow // B) == (hcol // HEAD_DIM)).astype(f32)
        qt = jnp.concatenate([q, q, q, q], axis=0) * hm
        scale = np.float32(1.0 / np.sqrt(HEAD_DIM))
        s_ref[...] = dot(qt, Kall.T) * scale
        wc_ref[...] = dot(W_out_ref[...], Wo_ref[...])
        bc_ref[...] = dot(bo_ref[...], W_out_ref[...].T) + b_out_ref[...]

    @pl.when(step == NCHUNK - 1)
    def _final():
        acc1 = s1_ref[...] + p1
        acc2 = s2_ref[...] + p2
        S1 = jnp.sum(acc1, axis=0, keepdims=True)
        S2 = jnp.sum(acc2, axis=0, keepdims=True)
        obs = obs_ref[...]
        dsq = S2 - 2.0 * obs * S1 + np.float32(NUM_MEMS) * (obs * obs)

        colf = jax.lax.broadcasted_iota(jnp.int32, (B, OBS_LEN), 1).astype(f32)
        mask = jnp.zeros((B, OBS_LEN), jnp.bool_)
        cur = dsq
        for _ in range(K):
            mval = jnp.min(cur, axis=1, keepdims=True)
            cand = jnp.where(cur == mval, colf, np.float32(512.0))
            midx = jnp.min(cand, axis=1, keepdims=True)
            sel = cand == midx
            mask = mask | sel
            cur = jnp.where(sel, _POS, cur)

        mask4 = jnp.concatenate([mask, mask, mask, mask], axis=0)
        s = jnp.where(mask4, s_ref[...], _NEG)
        m = jnp.max(s, axis=1, keepdims=True)
        e = jnp.where(mask4, jnp.exp(s - m), 0.0)
        w = e / jnp.sum(e, axis=1, keepdims=True)
        O = dot(w, vall_ref[...])
        hrow = jax.lax.broadcasted_iota(jnp.int32, (HEADS * B, EMBED), 0)
        hcol = jax.lax.broadcasted_iota(jnp.int32, (HEADS * B, EMBED), 1)
        hm = ((hrow // B) == (hcol // HEAD_DIM)).astype(f32)
        o = (O[0 * B:1 * B] * hm[0 * B:1 * B] + O[1 * B:2 * B] * hm[1 * B:2 * B]
             + O[2 * B:3 * B] * hm[2 * B:3 * B] + O[3 * B:4 * B] * hm[3 * B:4 * B])
        out_ref[...] = dot(o, wc_ref[...].T) + bc_ref[...]


@jax.jit
def kernel(obs, memories, W_obs, b_obs, Wq, bq, Wk, bk, Wv, bv,
           Wo, bo, W_out, b_out):
    biases = [b.reshape(1, -1) for b in (b_obs, bq, bk, bv, bo, b_out)]
    b_obs2, bq2, bk2, bv2, bo2, b_out2 = biases

    full = lambda shp: pl.BlockSpec(shp, lambda i: (0, 0))
    specs = [
        full((B, OBS_LEN)),
        pl.BlockSpec((CHUNK, OBS_LEN), lambda i: (i, 0)),
        pl.BlockSpec((256, MEM_LEN), lambda i: (0, 1)),
        full((EMBED, OBS_LEN)), full((1, EMBED)),
        full((EMBED, EMBED)), full((1, EMBED)),
        full((EMBED, MEM_LEN)), full((1, EMBED)),
        full((EMBED, MEM_LEN)), full((1, EMBED)),
        full((EMBED, EMBED)), full((1, EMBED)),
        full((OUT_DIM, EMBED)), full((1, OUT_DIM)),
    ]
    return pl.pallas_call(
        _body,
        grid=(NCHUNK,),
        in_specs=specs,
        out_specs=full((B, OUT_DIM)),
        out_shape=jax.ShapeDtypeStruct((B, OUT_DIM), jnp.float32),
        scratch_shapes=[pltpu.VMEM((8, OBS_LEN), jnp.float32),
                        pltpu.VMEM((8, OBS_LEN), jnp.float32),
                        pltpu.VMEM((256, EMBED), jnp.float32),
                        pltpu.VMEM((HEADS * B, OBS_LEN), jnp.float32),
                        pltpu.VMEM((OUT_DIM, EMBED), jnp.float32),
                        pltpu.VMEM((1, OUT_DIM), jnp.float32)],
    )(obs, memories, memories, W_obs, b_obs2, Wq, bq2, Wk, bk2,
      Wv, bv2, Wo, bo2, W_out, b_out2)


# speedup vs baseline: 1.0117x; 1.0117x over previous
import functools

import jax
import jax.numpy as jnp
import numpy as np
from jax.experimental import pallas as pl
from jax.experimental.pallas import tpu as pltpu

B = 128
OBS_LEN = 256
MEM_LEN = 256
NUM_MEMS = 4096
EMBED = 64
HEADS = 4
HEAD_DIM = EMBED // HEADS
K = 10
OUT_DIM = 64

_NEG = np.float32(-3.0e38)
_POS = np.float32(3.0e38)

NCHUNK = 2
CHUNK = NUM_MEMS // NCHUNK


def _body(obs_ref, memhead_ref, memtail_ref, W_obs_ref, b_obs_ref,
          Wq_ref, bq_ref, Wk_ref, bk_ref, Wv_ref, bv_ref,
          Wo_ref, bo_ref, W_out_ref, b_out_ref, out_ref,
          s1_ref, s2_ref, kall_ref, vall_ref, qt_ref):
    f32 = jnp.float32
    step = pl.program_id(0)
    dot = functools.partial(jnp.dot, preferred_element_type=f32)

    mh = memhead_ref[...]
    p1 = jnp.sum(mh.reshape(CHUNK // 8, 8, OBS_LEN), axis=0)
    p2 = jnp.sum((mh * mh).reshape(CHUNK // 8, 8, OBS_LEN), axis=0)

    @pl.when(step == 0)
    def _first():
        s1_ref[...] = p1
        s2_ref[...] = p2
        mt = memtail_ref[...]
        kall_ref[...] = dot(mt, Wk_ref[...].T) + bk_ref[...]
        vall_ref[...] = dot(mt, Wv_ref[...].T) + bv_ref[...]
        obs_emb = dot(obs_ref[...], W_obs_ref[...].T) + b_obs_ref[...]
        q = dot(obs_emb, Wq_ref[...].T) + bq_ref[...]
        hrow = jax.lax.broadcasted_iota(jnp.int32, (HEADS * B, EMBED), 0)
        hcol = jax.lax.broadcasted_iota(jnp.int32, (HEADS * B, EMBED), 1)
        hm = ((hrow // B) == (hcol // HEAD_DIM)).astype(f32)
        qt_ref[...] = jnp.concatenate([q, q, q, q], axis=0) * hm

    @pl.when(step == NCHUNK - 1)
    def _final():
        acc1 = s1_ref[...] + p1
        acc2 = s2_ref[...] + p2
        S1 = jnp.sum(acc1, axis=0, keepdims=True)
        S2 = jnp.sum(acc2, axis=0, keepdims=True)
        obs = obs_ref[...]
        dsq = S2 - 2.0 * obs * S1 + np.float32(NUM_MEMS) * (obs * obs)

        colf = jax.lax.broadcasted_iota(jnp.int32, (B, OBS_LEN), 1).astype(f32)
        mask = jnp.zeros((B, OBS_LEN), jnp.bool_)
        cur = dsq
        for _ in range(K):
            mval = jnp.min(cur, axis=1, keepdims=True)
            cand = jnp.where(cur == mval, colf, np.float32(512.0))
            midx = jnp.min(cand, axis=1, keepdims=True)
            sel = cand == midx
            mask = mask | sel
            cur = jnp.where(sel, _POS, cur)

        scale = np.float32(1.0 / np.sqrt(HEAD_DIM))
        Kall = kall_ref[...]
        Vall = vall_ref[...]
        qt = qt_ref[...]
        s = dot(qt, Kall.T) * scale
        mask4 = jnp.concatenate([mask, mask, mask, mask], axis=0)
        s = jnp.where(mask4, s, _NEG)
        m = jnp.max(s, axis=1, keepdims=True)
        e = jnp.where(mask4, jnp.exp(s - m), 0.0)
        w = e / jnp.sum(e, axis=1, keepdims=True)
        O = dot(w, Vall)
        hrow = jax.lax.broadcasted_iota(jnp.int32, (HEADS * B, EMBED), 0)
        hcol = jax.lax.broadcasted_iota(jnp.int32, (HEADS * B, EMBED), 1)
        hm = ((hrow // B) == (hcol // HEAD_DIM)).astype(f32)
        o = (O[0 * B:1 * B] * hm[0 * B:1 * B] + O[1 * B:2 * B] * hm[1 * B:2 * B]
             + O[2 * B:3 * B] * hm[2 * B:3 * B] + O[3 * B:4 * B] * hm[3 * B:4 * B])
        feat = dot(o, Wo_ref[...].T) + bo_ref[...]
        out_ref[...] = dot(feat, W_out_ref[...].T) + b_out_ref[...]


@jax.jit
def kernel(obs, memories, W_obs, b_obs, Wq, bq, Wk, bk, Wv, bv,
           Wo, bo, W_out, b_out):
    biases = [b.reshape(1, -1) for b in (b_obs, bq, bk, bv, bo, b_out)]
    b_obs2, bq2, bk2, bv2, bo2, b_out2 = biases

    full = lambda shp: pl.BlockSpec(shp, lambda i: (0, 0))
    specs = [
        full((B, OBS_LEN)),
        pl.BlockSpec((CHUNK, OBS_LEN), lambda i: (i, 0)),
        pl.BlockSpec((256, MEM_LEN), lambda i: (0, 1)),
        full((EMBED, OBS_LEN)), full((1, EMBED)),
        full((EMBED, EMBED)), full((1, EMBED)),
        full((EMBED, MEM_LEN)), full((1, EMBED)),
        full((EMBED, MEM_LEN)), full((1, EMBED)),
        full((EMBED, EMBED)), full((1, EMBED)),
        full((OUT_DIM, EMBED)), full((1, OUT_DIM)),
    ]
    return pl.pallas_call(
        _body,
        grid=(NCHUNK,),
        in_specs=specs,
        out_specs=full((B, OUT_DIM)),
        out_shape=jax.ShapeDtypeStruct((B, OUT_DIM), jnp.float32),
        scratch_shapes=[pltpu.VMEM((8, OBS_LEN), jnp.float32),
                        pltpu.VMEM((8, OBS_LEN), jnp.float32),
                        pltpu.VMEM((256, EMBED), jnp.float32),
                        pltpu.VMEM((256, EMBED), jnp.float32),
                        pltpu.VMEM((HEADS * B, EMBED), jnp.float32)],
    )(obs, memories, memories, W_obs, b_obs2, Wq, bq2, Wk, bk2,
      Wv, bv2, Wo, bo2, W_out, b_out2)
